# baseline (speedup 1.0000x reference)
import functools

import jax
import jax.numpy as jnp
from jax import lax
from jax.experimental import pallas as pl
import jax.experimental.pallas.tpu as pltpu
from jax.experimental.pallas import tpu_sc as plsc

LENGTH = 2048
CHANNEL = 1024
N_SEL = 8
NUM_HEADS = 4
NUM_FREQ = 16
BATCH = 4
CHUNK = CHANNEL // N_SEL

ROWS = 1024


def _sc_selector(selw_hbm, hw_hbm, coeff_hbm,
                 selw_v, hw_v, perm_v, vals_v, out_v):
    info = plsc.get_sparse_core_info()
    nc = info.num_cores
    wid = lax.axis_index("s") * nc + lax.axis_index("c")

    @pl.when(wid < N_SEL)
    def _():
        k = wid

        hw_v[...] = jnp.zeros((NUM_FREQ,), jnp.float32)
        pltpu.sync_copy(hw_hbm, hw_v.at[pl.ds(0, NUM_HEADS)])
        pltpu.sync_copy(selw_hbm, selw_v)

        iota16 = lax.iota(jnp.int32, 16)
        kk = jnp.full((16,), k, jnp.int32)

        hwv = hw_v[...]
        m = iota16 < NUM_HEADS
        hmax = jnp.max(jnp.where(m, hwv, jnp.float32(-1e30)), axis=0)
        he = jnp.where(m, jnp.exp(hwv - hmax), jnp.float32(0.0))
        hw_sm = he / jnp.sum(he, axis=0)

        acc = jnp.zeros((16,), jnp.float32)
        for h in range(NUM_HEADS):
            logits = selw_v[h] * jnp.float32(h + 1)
            mx = jnp.max(logits, axis=0)
            e = jnp.exp(logits - mx)
            nw = e / jnp.sum(e, axis=0)
            vals, perm = plsc.sort_key_val(nw, iota16, descending=True)
            perm_v[...] = perm
            vals_v[...] = vals
            idxsplat = plsc.load_gather(perm_v, [kk])
            vsplat = plsc.load_gather(vals_v, [kk])
            hscal = jnp.sum(jnp.where(iota16 == h, hw_sm, jnp.float32(0.0)),
                            axis=0)
            onehot = jnp.where(iota16 == idxsplat, jnp.float32(1.0),
                               jnp.float32(0.0))
            acc = acc + hscal * vsplat * onehot
        out_v[...] = acc
        pltpu.sync_copy(out_v, coeff_hbm.at[k])


def _selector_coeff(sel_weights, head_weights):
    mesh = plsc.VectorSubcoreMesh(core_axis_name="c", subcore_axis_name="s")
    kfn = functools.partial(
        pl.kernel,
        mesh=mesh,
        out_type=jax.ShapeDtypeStruct((N_SEL, NUM_FREQ), jnp.float32),
        scratch_types=[
            pltpu.VMEM((NUM_HEADS, NUM_FREQ), jnp.float32),
            pltpu.VMEM((NUM_FREQ,), jnp.float32),
            pltpu.VMEM((NUM_FREQ,), jnp.int32),
            pltpu.VMEM((NUM_FREQ,), jnp.float32),
            pltpu.VMEM((NUM_FREQ,), jnp.float32),
        ],
        compiler_params=pltpu.CompilerParams(needs_layout_passes=False),
    )(_sc_selector)
    return kfn(sel_weights, head_weights)


def _reduce_kernel(coeff_ref, x_ref, base_ref, out_ref, w_scratch):
    b = pl.program_id(0)
    kblk = pl.program_id(1)

    @pl.when(jnp.logical_and(b == 0, kblk == 0))
    def _expand_w():
        w_scratch[...] = jnp.dot(coeff_ref[...], base_ref[...],
                                 preferred_element_type=jnp.float32)

    for j in range(ROWS // CHUNK):
        wrow = w_scratch[kblk * (ROWS // CHUNK) + j, :]
        xsub = x_ref[0, pl.ds(j * CHUNK, CHUNK), :]
        out_ref[0, 0, 0, pl.ds(j * CHUNK, CHUNK)] = jnp.sum(
            xsub * wrow[None, :], axis=1)


@jax.jit
def kernel(x, sel_weights, head_weights, base_weight):
    coeff = _selector_coeff(sel_weights, head_weights)
    out = pl.pallas_call(
        _reduce_kernel,
        grid=(BATCH, CHANNEL // ROWS),
        in_specs=[
            pl.BlockSpec((N_SEL, NUM_FREQ), lambda b, k: (0, 0)),
            pl.BlockSpec((1, ROWS, LENGTH), lambda b, k: (b, k, 0)),
            pl.BlockSpec((NUM_FREQ, LENGTH), lambda b, k: (0, 0)),
        ],
        out_specs=pl.BlockSpec((1, 1, 1, ROWS), lambda b, k: (b, k, 0, 0)),
        out_shape=jax.ShapeDtypeStruct((BATCH, CHANNEL // ROWS, 1, ROWS),
                                       jnp.float32),
        scratch_shapes=[pltpu.VMEM((N_SEL, LENGTH), jnp.float32)],
    )(coeff, x, base_weight)
    return out.reshape(BATCH, CHANNEL)


# speedup vs baseline: 1.0458x; 1.0458x over previous
import functools

import jax
import jax.numpy as jnp
from jax import lax
from jax.experimental import pallas as pl
import jax.experimental.pallas.tpu as pltpu
from jax.experimental.pallas import tpu_sc as plsc

LENGTH = 2048
CHANNEL = 1024
N_SEL = 8
NUM_HEADS = 4
NUM_FREQ = 16
BATCH = 4
CHUNK = CHANNEL // N_SEL

ROWS = 1024


def _sc_selector(selw_hbm, hw_hbm, coeff_hbm,
                 selw_v, hw_v, perm_v, vals_v, out_v):
    info = plsc.get_sparse_core_info()
    nc = info.num_cores
    wid = lax.axis_index("s") * nc + lax.axis_index("c")

    @pl.when(wid < N_SEL)
    def _():
        k = wid

        hw_v[...] = jnp.zeros((NUM_FREQ,), jnp.float32)
        pltpu.sync_copy(hw_hbm, hw_v.at[pl.ds(0, NUM_HEADS)])
        pltpu.sync_copy(selw_hbm, selw_v)

        iota16 = lax.iota(jnp.int32, 16)
        kk = jnp.full((16,), k, jnp.int32)

        hwv = hw_v[...]
        m = iota16 < NUM_HEADS
        hmax = jnp.max(jnp.where(m, hwv, jnp.float32(-1e30)), axis=0)
        he = jnp.where(m, jnp.exp(hwv - hmax), jnp.float32(0.0))
        hw_sm = he / jnp.sum(he, axis=0)

        acc = jnp.zeros((16,), jnp.float32)
        for h in range(NUM_HEADS):
            logits = selw_v[h] * jnp.float32(h + 1)
            mx = jnp.max(logits, axis=0)
            e = jnp.exp(logits - mx)
            nw = e / jnp.sum(e, axis=0)
            vals, perm = plsc.sort_key_val(nw, iota16, descending=True)
            perm_v[...] = perm
            vals_v[...] = vals
            idxsplat = plsc.load_gather(perm_v, [kk])
            vsplat = plsc.load_gather(vals_v, [kk])
            hscal = jnp.sum(jnp.where(iota16 == h, hw_sm, jnp.float32(0.0)),
                            axis=0)
            onehot = jnp.where(iota16 == idxsplat, jnp.float32(1.0),
                               jnp.float32(0.0))
            acc = acc + hscal * vsplat * onehot
        out_v[...] = acc
        pltpu.sync_copy(out_v, coeff_hbm.at[k])


def _selector_coeff(sel_weights, head_weights):
    mesh = plsc.VectorSubcoreMesh(core_axis_name="c", subcore_axis_name="s",
                                  num_cores=1)
    kfn = functools.partial(
        pl.kernel,
        mesh=mesh,
        out_type=jax.ShapeDtypeStruct((N_SEL, NUM_FREQ), jnp.float32),
        scratch_types=[
            pltpu.VMEM((NUM_HEADS, NUM_FREQ), jnp.float32),
            pltpu.VMEM((NUM_FREQ,), jnp.float32),
            pltpu.VMEM((NUM_FREQ,), jnp.int32),
            pltpu.VMEM((NUM_FREQ,), jnp.float32),
            pltpu.VMEM((NUM_FREQ,), jnp.float32),
        ],
        compiler_params=pltpu.CompilerParams(needs_layout_passes=False),
    )(_sc_selector)
    return kfn(sel_weights, head_weights)


def _reduce_kernel(coeff_ref, x_ref, base_ref, out_ref, w_scratch):
    b = pl.program_id(0)
    kblk = pl.program_id(1)

    @pl.when(jnp.logical_and(b == 0, kblk == 0))
    def _expand_w():
        w_scratch[...] = jnp.dot(coeff_ref[...], base_ref[...],
                                 preferred_element_type=jnp.float32)

    for j in range(ROWS // CHUNK):
        wrow = w_scratch[kblk * (ROWS // CHUNK) + j, :]
        xsub = x_ref[0, pl.ds(j * CHUNK, CHUNK), :]
        out_ref[0, 0, 0, pl.ds(j * CHUNK, CHUNK)] = jnp.sum(
            xsub * wrow[None, :], axis=1)


@jax.jit
def kernel(x, sel_weights, head_weights, base_weight):
    coeff = _selector_coeff(sel_weights, head_weights)
    out = pl.pallas_call(
        _reduce_kernel,
        grid=(BATCH, CHANNEL // ROWS),
        in_specs=[
            pl.BlockSpec((N_SEL, NUM_FREQ), lambda b, k: (0, 0)),
            pl.BlockSpec((1, ROWS, LENGTH), lambda b, k: (b, k, 0)),
            pl.BlockSpec((NUM_FREQ, LENGTH), lambda b, k: (0, 0)),
        ],
        out_specs=pl.BlockSpec((1, 1, 1, ROWS), lambda b, k: (b, k, 0, 0)),
        out_shape=jax.ShapeDtypeStruct((BATCH, CHANNEL // ROWS, 1, ROWS),
                                       jnp.float32),
        scratch_shapes=[pltpu.VMEM((N_SEL, LENGTH), jnp.float32)],
    )(coeff, x, base_weight)
    return out.reshape(BATCH, CHANNEL)
